# baseline (device time: 28144 ns/iter reference)
import os

import jax
import jax.numpy as jnp
from jax import lax
from jax.experimental import pallas as pl
from jax.experimental.pallas import tpu as pltpu

_BENCH = os.environ.get("BENCH", "")

N_DEV = 4
B, SQ, D = 4, 256, 1024
DH = 128
SCALE = 0.08838834764831843
ROWS = B * SQ
U = 128


def kernel(x, Wq, Wo, Wk, Wv):
    n_heads = Wq.shape[1] // DH
    x2 = x.reshape(ROWS, D)

    def body(x_ref, wq_ref, wo_ref, wk_ref, wv_ref, out_ref,
             sbuf_ref, rbuf_ref, send_sems, recv_sems):
        my = lax.axis_index("i")
        a_half = (my ^ (my >> 1)) & 1
        b_half = (my >> 1) & 1
        qa_sub = b_half
        qb_sub = my & 1
        p1 = my ^ 1
        p2 = 3 - my

        barrier_sem = pltpu.get_barrier_semaphore()
        for nbr in (p1, p2):
            pl.semaphore_signal(
                barrier_sem, inc=1,
                device_id=(nbr,), device_id_type=pl.DeviceIdType.MESH,
            )
        pl.semaphore_wait(barrier_sem, 2)

        wq16 = wq_ref[...].astype(jnp.bfloat16)
        wk16 = wk_ref[...].astype(jnp.bfloat16)
        wv16 = wv_ref[...].astype(jnp.bfloat16)
        wo16 = wo_ref[...].astype(jnp.bfloat16)

        xv16 = x_ref[...].astype(jnp.bfloat16)
        qfull = jnp.dot(xv16, wq16, preferred_element_type=jnp.float32)
        kfull = jnp.dot(xv16, wk16, preferred_element_type=jnp.float32)
        vfull = jnp.dot(xv16, wv16, preferred_element_type=jnp.float32)

        def compute_batch(bi):
            r0 = bi * SQ
            r = pl.ds(r0, SQ)
            qm = qfull[r0:r0 + SQ, :]
            km = kfull[r0:r0 + SQ, :]
            vm = vfull[r0:r0 + SQ, :]
            outs = []
            for h in range(n_heads):
                c0 = h * DH
                qh = qm[:, c0:c0 + DH]
                kh = km[:, c0:c0 + DH]
                vh = vm[:, c0:c0 + DH]
                s = lax.dot_general(
                    qh, kh, (((1,), (1,)), ((), ())),
                    preferred_element_type=jnp.float32,
                ) * SCALE
                m = jnp.max(s, axis=-1, keepdims=True)
                p = jnp.exp(s - m)
                l = jnp.sum(p, axis=-1, keepdims=True)
                outs.append(
                    jnp.dot(p, vh, preferred_element_type=jnp.float32) / l
                )
            attn_b = jnp.concatenate(outs, axis=1).astype(jnp.bfloat16)
            out_ref[r, :] = jnp.dot(
                attn_b, wo16, preferred_element_type=jnp.float32
            )

        def exch(slot, n_u, partner, sem):
            rdma = pltpu.make_async_remote_copy(
                src_ref=sbuf_ref.at[slot, pl.ds(0, n_u * U), :],
                dst_ref=rbuf_ref.at[slot, pl.ds(0, n_u * U), :],
                send_sem=send_sems.at[sem],
                recv_sem=recv_sems.at[sem],
                device_id=(partner,),
                device_id_type=pl.DeviceIdType.MESH,
            )
            rdma.start()
            return rdma

        def stage(slot, src_u, n_u):
            rows = pl.ds(src_u * U, n_u * U)
            sbuf_ref[slot, pl.ds(0, n_u * U), :] = (
                out_ref[rows, :].astype(jnp.bfloat16)
            )

        def accum(slot, dst_u, n_u):
            rows = pl.ds(dst_u * U, n_u * U)
            out_ref[rows, :] = out_ref[rows, :] + (
                rbuf_ref[slot, pl.ds(0, n_u * U), :].astype(jnp.float32)
            )

        def store(slot, dst_u, n_u):
            rows = pl.ds(dst_u * U, n_u * U)
            out_ref[rows, :] = (
                rbuf_ref[slot, pl.ds(0, n_u * U), :].astype(jnp.float32)
            )

        if _BENCH == "compute":
            for bi in range(B):
                compute_batch(bi)
            return
        if _BENCH == "matmul":
            for bi in range(B):
                r = pl.ds(bi * SQ, SQ)
                xb = x_ref[r, :].astype(jnp.bfloat16)
                qm = jnp.dot(xb, wq16, preferred_element_type=jnp.float32)
                km = jnp.dot(xb, wk16, preferred_element_type=jnp.float32)
                vm = jnp.dot(xb, wv16, preferred_element_type=jnp.float32)
                acc = (qm + km + vm).astype(jnp.bfloat16)
                out_ref[r, :] = jnp.dot(
                    acc, wo16, preferred_element_type=jnp.float32
                )
            return
        if _BENCH == "comm":
            out_ref[...] = jnp.zeros((ROWS, D), jnp.float32)
            stage(0, 2 * (1 - a_half), 2)
            ra = exch(0, 2, p1, 0)
            stage(1, 4 + 2 * (1 - b_half), 2)
            rb = exch(1, 2, p2, 1)
        else:
            compute_batch(0)
            compute_batch(1)
            stage(0, 2 * (1 - a_half), 2)
            ra = exch(0, 2, p1, 0)
            compute_batch(2)
            compute_batch(3)
            stage(1, 4 + 2 * (1 - b_half), 2)
            rb = exch(1, 2, p2, 1)

        qa = 2 * a_half + qa_sub
        qa_p = 2 * a_half + (1 - qa_sub)
        qb_ = 4 + 2 * b_half + qb_sub
        qb_p = 4 + 2 * b_half + (1 - qb_sub)

        ra.wait()
        accum(0, 2 * a_half, 2)
        stage(2, qa_p, 1)
        ra = exch(2, 1, p2, 2)
        rb.wait()
        accum(1, 4 + 2 * b_half, 2)
        stage(3, qb_p, 1)
        rb = exch(3, 1, p1, 3)

        ra.wait()
        accum(2, qa, 1)
        stage(4, qa, 1)
        ra = exch(4, 1, p2, 4)
        rb.wait()
        accum(3, qb_, 1)
        stage(5, qb_, 1)
        rb = exch(5, 1, p1, 5)

        ra.wait()
        store(4, qa_p, 1)
        stage(6, 2 * a_half, 2)
        ra = exch(6, 2, p1, 6)
        rb.wait()
        store(5, qb_p, 1)
        stage(7, 4 + 2 * b_half, 2)
        rb = exch(7, 2, p2, 7)

        ra.wait()
        store(6, 2 * (1 - a_half), 2)
        rb.wait()
        store(7, 4 + 2 * (1 - b_half), 2)

    out2 = pl.pallas_call(
        body,
        out_shape=jax.ShapeDtypeStruct((ROWS, D), jnp.float32),
        in_specs=[pl.BlockSpec(memory_space=pltpu.VMEM)] * 5,
        out_specs=pl.BlockSpec(memory_space=pltpu.VMEM),
        scratch_shapes=[
            pltpu.VMEM((8, 2 * U, D), jnp.bfloat16),
            pltpu.VMEM((8, 2 * U, D), jnp.bfloat16),
            pltpu.SemaphoreType.DMA((8,)),
            pltpu.SemaphoreType.DMA((8,)),
        ],
        compiler_params=pltpu.CompilerParams(collective_id=0),
    )(x2, Wq, Wo, Wk, Wv)
    return out2.reshape(B, SQ, D)


# device time: 18569 ns/iter; 1.5156x vs baseline; 1.5156x over previous
import os

import jax
import jax.numpy as jnp
from jax import lax
from jax.experimental import pallas as pl
from jax.experimental.pallas import tpu as pltpu

_BENCH = os.environ.get("BENCH", "")

N_DEV = 4
B, SQ, D = 4, 256, 1024
DH = 128
SCALE = 0.08838834764831843
ROWS = B * SQ
U = 128


def kernel(x, Wq, Wo, Wk, Wv):
    n_heads = Wq.shape[1] // DH
    x2 = x.reshape(ROWS, D)

    def body(x_ref, wq_ref, wo_ref, wk_ref, wv_ref, out_ref,
             sbuf_ref, rbuf_ref, send_sems, recv_sems):
        my = lax.axis_index("i")
        a_half = (my ^ (my >> 1)) & 1
        b_half = (my >> 1) & 1
        qa_sub = b_half
        qb_sub = my & 1
        p1 = my ^ 1
        p2 = 3 - my

        barrier_sem = pltpu.get_barrier_semaphore()
        for nbr in (p1, p2):
            pl.semaphore_signal(
                barrier_sem, inc=1,
                device_id=(nbr,), device_id_type=pl.DeviceIdType.MESH,
            )
        pl.semaphore_wait(barrier_sem, 2)

        wq16 = wq_ref[...].astype(jnp.bfloat16)
        wk16 = wk_ref[...].astype(jnp.bfloat16)
        wv16 = wv_ref[...].astype(jnp.bfloat16)
        wo16 = wo_ref[...].astype(jnp.bfloat16)

        xv16 = x_ref[...].astype(jnp.bfloat16)
        qfull = jnp.dot(xv16, wq16, preferred_element_type=jnp.float32)
        kfull = jnp.dot(xv16, wk16, preferred_element_type=jnp.float32)
        vfull = jnp.dot(xv16, wv16, preferred_element_type=jnp.float32)

        def compute_batch(bi):
            r0 = bi * SQ
            r = pl.ds(r0, SQ)
            qm = qfull[r0:r0 + SQ, :]
            km = kfull[r0:r0 + SQ, :]
            vm = vfull[r0:r0 + SQ, :]
            outs = []
            for h in range(n_heads):
                c0 = h * DH
                qh = qm[:, c0:c0 + DH]
                kh = km[:, c0:c0 + DH]
                vh = vm[:, c0:c0 + DH]
                s = lax.dot_general(
                    qh, kh, (((1,), (1,)), ((), ())),
                    preferred_element_type=jnp.float32,
                ) * SCALE
                m = jnp.max(s, axis=-1, keepdims=True)
                p = jnp.exp(s - m)
                l = jnp.sum(p, axis=-1, keepdims=True)
                outs.append(
                    jnp.dot(p, vh, preferred_element_type=jnp.float32) / l
                )
            attn_b = jnp.concatenate(outs, axis=1).astype(jnp.bfloat16)
            out_ref[r, :] = jnp.dot(
                attn_b, wo16, preferred_element_type=jnp.float32
            )

        def exch(slot, n_u, partner, sem):
            rdma = pltpu.make_async_remote_copy(
                src_ref=sbuf_ref.at[slot, pl.ds(0, n_u * U), :],
                dst_ref=rbuf_ref.at[slot, pl.ds(0, n_u * U), :],
                send_sem=send_sems.at[sem],
                recv_sem=recv_sems.at[sem],
                device_id=(partner,),
                device_id_type=pl.DeviceIdType.MESH,
            )
            rdma.start()
            return rdma

        def stage(slot, src_u, n_u):
            rows = pl.ds(src_u * U, n_u * U)
            sbuf_ref[slot, pl.ds(0, n_u * U), :] = (
                out_ref[rows, :].astype(jnp.bfloat16)
            )

        def accum(slot, dst_u, n_u):
            rows = pl.ds(dst_u * U, n_u * U)
            out_ref[rows, :] = out_ref[rows, :] + (
                rbuf_ref[slot, pl.ds(0, n_u * U), :].astype(jnp.float32)
            )

        def store(slot, dst_u, n_u):
            rows = pl.ds(dst_u * U, n_u * U)
            out_ref[rows, :] = (
                rbuf_ref[slot, pl.ds(0, n_u * U), :].astype(jnp.float32)
            )

        if _BENCH == "compute":
            for bi in range(B):
                compute_batch(bi)
            return
        if _BENCH == "matmul":
            f8 = jnp.float8_e4m3fn
            wq8 = (wq_ref[...] * 50.0).astype(f8)
            wk8 = (wk_ref[...] * 50.0).astype(f8)
            wv8 = (wv_ref[...] * 50.0).astype(f8)
            wo8 = (wo_ref[...] * 50.0).astype(f8)
            for bi in range(B):
                r = pl.ds(bi * SQ, SQ)
                xb = x_ref[r, :].astype(f8)
                qm = jnp.dot(xb, wq8, preferred_element_type=jnp.float32)
                km = jnp.dot(xb, wk8, preferred_element_type=jnp.float32)
                vm = jnp.dot(xb, wv8, preferred_element_type=jnp.float32)
                acc = ((qm + km + vm) * 0.02).astype(f8)
                out_ref[r, :] = jnp.dot(
                    acc, wo8, preferred_element_type=jnp.float32
                )
            return
        if _BENCH == "comm":
            out_ref[...] = jnp.zeros((ROWS, D), jnp.float32)
            stage(0, 2 * (1 - a_half), 2)
            ra = exch(0, 2, p1, 0)
            stage(1, 4 + 2 * (1 - b_half), 2)
            rb = exch(1, 2, p2, 1)
        else:
            compute_batch(0)
            compute_batch(1)
            stage(0, 2 * (1 - a_half), 2)
            ra = exch(0, 2, p1, 0)
            compute_batch(2)
            compute_batch(3)
            stage(1, 4 + 2 * (1 - b_half), 2)
            rb = exch(1, 2, p2, 1)

        qa = 2 * a_half + qa_sub
        qa_p = 2 * a_half + (1 - qa_sub)
        qb_ = 4 + 2 * b_half + qb_sub
        qb_p = 4 + 2 * b_half + (1 - qb_sub)

        ra.wait()
        accum(0, 2 * a_half, 2)
        stage(2, qa_p, 1)
        ra = exch(2, 1, p2, 2)
        rb.wait()
        accum(1, 4 + 2 * b_half, 2)
        stage(3, qb_p, 1)
        rb = exch(3, 1, p1, 3)

        ra.wait()
        accum(2, qa, 1)
        stage(4, qa, 1)
        ra = exch(4, 1, p2, 4)
        rb.wait()
        accum(3, qb_, 1)
        stage(5, qb_, 1)
        rb = exch(5, 1, p1, 5)

        ra.wait()
        store(4, qa_p, 1)
        stage(6, 2 * a_half, 2)
        ra = exch(6, 2, p1, 6)
        rb.wait()
        store(5, qb_p, 1)
        stage(7, 4 + 2 * b_half, 2)
        rb = exch(7, 2, p2, 7)

        ra.wait()
        store(6, 2 * (1 - a_half), 2)
        rb.wait()
        store(7, 4 + 2 * (1 - b_half), 2)

    out2 = pl.pallas_call(
        body,
        out_shape=jax.ShapeDtypeStruct((ROWS, D), jnp.float32),
        in_specs=[pl.BlockSpec(memory_space=pltpu.VMEM)] * 5,
        out_specs=pl.BlockSpec(memory_space=pltpu.VMEM),
        scratch_shapes=[
            pltpu.VMEM((8, 2 * U, D), jnp.bfloat16),
            pltpu.VMEM((8, 2 * U, D), jnp.bfloat16),
            pltpu.SemaphoreType.DMA((8,)),
            pltpu.SemaphoreType.DMA((8,)),
        ],
        compiler_params=pltpu.CompilerParams(collective_id=0),
    )(x2, Wq, Wo, Wk, Wv)
    return out2.reshape(B, SQ, D)
